# baseline (speedup 1.0000x reference)
import jax
import jax.numpy as jnp
from jax import lax
from jax.experimental import pallas as pl
from jax.experimental.pallas import tpu as pltpu
from jax.experimental.pallas import tpu_sc as plsc

BATCH = 16384
EMBED_DIM = 64
NUM_ROWS = 1000000
CHUNK = 128
NBUF = 4

EBLK = 1000
NBLOCKS = NUM_ROWS // EBLK
HALF = EBLK // 2

_info = plsc.get_sparse_core_info()
NUM_CORES = _info.num_cores
NUM_SUBCORES = _info.num_subcores
NUM_WORKERS = NUM_CORES * NUM_SUBCORES
ROWS_PER_WORKER = BATCH // NUM_WORKERS
SET_CHUNKS = ROWS_PER_WORKER // CHUNK
TOTAL_CHUNKS = 2 * SET_CHUNKS
BLOCKS_PER_WORKER = -(-NBLOCKS // NUM_WORKERS)


def _convert_kernel(entT, relT, ent_out, rel_out, inbuf, outbuf):
    wid = lax.axis_index("s") * NUM_CORES + lax.axis_index("c")
    iota = lax.iota(jnp.int32, 16)
    idx_d = [iota + 16 * k for k in range(EMBED_DIM // 16)]

    for src, dst in ((entT, ent_out), (relT, rel_out)):
        def block_body(t, carry):
            g = wid + NUM_WORKERS * t

            @pl.when(g < NBLOCKS)
            def _():
                e0 = g * EBLK
                pltpu.sync_copy(src.at[:, pl.ds(e0, EBLK)], inbuf)
                for half in range(2):
                    def el_body(el_rel, carry2):
                        el = half * HALF + el_rel
                        idx_e = jnp.full((16,), el, jnp.int32)
                        for k in range(EMBED_DIM // 16):
                            v = plsc.load_gather(inbuf, [idx_d[k], idx_e])
                            outbuf[pl.ds(el_rel * EMBED_DIM + 16 * k, 16)] = v
                        return carry2

                    lax.fori_loop(0, HALF, el_body, 0, unroll=2)
                    pltpu.sync_copy(
                        outbuf,
                        dst.at[pl.ds((e0 + half * HALF) * EMBED_DIM,
                                     HALF * EMBED_DIM)])
            return carry

        lax.fori_loop(0, BLOCKS_PER_WORKER, block_body, 0)


def _gather_kernel(ent_hbm, rel_hbm,
                   ph_hbm, pr_hbm, pt_hbm, nh_hbm, nr_hbm, nt_hbm,
                   pos_out, neg_out,
                   ih_v, ir_v, it_v, h_v, r_v, t_v,
                   sem0, sem1, sem2, sem3):
    wid = lax.axis_index("s") * NUM_CORES + lax.axis_index("c")
    wbase = wid * ROWS_PER_WORKER
    sems = [sem0, sem1, sem2, sem3]

    pltpu.sync_copy(ph_hbm.at[wid], ih_v.at[pl.ds(0, SET_CHUNKS)])
    pltpu.sync_copy(pr_hbm.at[wid], ir_v.at[pl.ds(0, SET_CHUNKS)])
    pltpu.sync_copy(pt_hbm.at[wid], it_v.at[pl.ds(0, SET_CHUNKS)])
    pltpu.sync_copy(nh_hbm.at[wid], ih_v.at[pl.ds(SET_CHUNKS, SET_CHUNKS)])
    pltpu.sync_copy(nr_hbm.at[wid], ir_v.at[pl.ds(SET_CHUNKS, SET_CHUNKS)])
    pltpu.sync_copy(nt_hbm.at[wid], it_v.at[pl.ds(SET_CHUNKS, SET_CHUNKS)])

    def fire(g, s):
        sem = sems[s]
        sl = pl.ds(s * CHUNK, CHUNK)
        return (
            pltpu.async_copy(ent_hbm.at[ih_v.at[g]], h_v.at[sl], sem),
            pltpu.async_copy(rel_hbm.at[ir_v.at[g]], r_v.at[sl], sem),
            pltpu.async_copy(ent_hbm.at[it_v.at[g]], t_v.at[sl], sem),
        )

    inflight = {}
    for g in range(NBUF):
        inflight[g] = fire(g, g % NBUF)

    for g in range(TOTAL_CHUNKS):
        s = g % NBUF
        for cp in inflight.pop(g):
            cp.wait()
        base = s * CHUNK

        def row_body(i, carry):
            for k in range(EMBED_DIM // 16):
                sl = pl.ds(k * 16, 16)
                h_v[base + i, sl] = jnp.abs(
                    h_v[base + i, sl] + r_v[base + i, sl] - t_v[base + i, sl])
            return carry

        lax.fori_loop(0, CHUNK, row_body, 0, unroll=4)

        out_hbm = pos_out if g < SET_CHUNKS else neg_out
        row0 = wbase + (g % SET_CHUNKS) * CHUNK
        pltpu.sync_copy(h_v.at[pl.ds(base, CHUNK)],
                        out_hbm.at[pl.ds(row0, CHUNK)])
        if g + NBUF < TOTAL_CHUNKS:
            inflight[g + NBUF] = fire(g + NBUF, s)


@jax.jit
def kernel(positive_samples, negative_samples, entity_embedding, relation_embedding):
    mesh = plsc.VectorSubcoreMesh(core_axis_name="c", subcore_axis_name="s")
    lin_t = jax.ShapeDtypeStruct((NUM_ROWS * EMBED_DIM,), jnp.float32)
    convert = pl.kernel(
        _convert_kernel,
        out_type=(lin_t, lin_t),
        mesh=mesh,
        compiler_params=pltpu.CompilerParams(
            use_tc_tiling_on_sc=False, needs_layout_passes=False),
        scratch_types=[
            pltpu.VMEM((EMBED_DIM, EBLK), jnp.float32),
            pltpu.VMEM((HALF * EMBED_DIM,), jnp.float32),
        ],
    )
    ent_lin, rel_lin = convert(entity_embedding.T, relation_embedding.T)
    ent_lin = ent_lin.reshape(NUM_ROWS, EMBED_DIM)
    rel_lin = rel_lin.reshape(NUM_ROWS, EMBED_DIM)

    idx_shape = (NUM_WORKERS, SET_CHUNKS, CHUNK)
    ph = positive_samples[:, 0].reshape(idx_shape)
    pr = positive_samples[:, 1].reshape(idx_shape)
    pt = positive_samples[:, 2].reshape(idx_shape)
    nh = negative_samples[:, 0].reshape(idx_shape)
    nr = negative_samples[:, 1].reshape(idx_shape)
    nt = negative_samples[:, 2].reshape(idx_shape)

    out_t = jax.ShapeDtypeStruct((BATCH, EMBED_DIM), jnp.float32)
    gather = pl.kernel(
        _gather_kernel,
        out_type=(out_t, out_t),
        mesh=mesh,
        compiler_params=pltpu.CompilerParams(use_tc_tiling_on_sc=False),
        scratch_types=[
            pltpu.VMEM((TOTAL_CHUNKS, CHUNK), jnp.int32),
            pltpu.VMEM((TOTAL_CHUNKS, CHUNK), jnp.int32),
            pltpu.VMEM((TOTAL_CHUNKS, CHUNK), jnp.int32),
            pltpu.VMEM((NBUF * CHUNK, EMBED_DIM), jnp.float32),
            pltpu.VMEM((NBUF * CHUNK, EMBED_DIM), jnp.float32),
            pltpu.VMEM((NBUF * CHUNK, EMBED_DIM), jnp.float32),
            pltpu.SemaphoreType.DMA,
            pltpu.SemaphoreType.DMA,
            pltpu.SemaphoreType.DMA,
            pltpu.SemaphoreType.DMA,
        ],
    )
    pos_out, neg_out = gather(ent_lin, rel_lin, ph, pr, pt, nh, nr, nt)
    return pos_out, neg_out


# speedup vs baseline: 10.1986x; 10.1986x over previous
import jax
import jax.numpy as jnp
from jax import lax
from jax.experimental import pallas as pl
from jax.experimental.pallas import tpu as pltpu
from jax.experimental.pallas import tpu_sc as plsc

BATCH = 16384
EMBED_DIM = 64
CHUNK = 128
NBUF = 4

_info = plsc.get_sparse_core_info()
NUM_CORES = _info.num_cores
NUM_SUBCORES = _info.num_subcores
NUM_WORKERS = NUM_CORES * NUM_SUBCORES
ROWS_PER_WORKER = BATCH // NUM_WORKERS
SET_CHUNKS = ROWS_PER_WORKER // CHUNK
TOTAL_CHUNKS = 2 * SET_CHUNKS


def _transe_kernel(ent_hbm, rel_hbm,
                   ph_hbm, pr_hbm, pt_hbm, nh_hbm, nr_hbm, nt_hbm,
                   pos_out, neg_out,
                   ih_v, ir_v, it_v, h_v, r_v, t_v,
                   sem0, sem1, sem2, sem3):
    wid = lax.axis_index("s") * NUM_CORES + lax.axis_index("c")
    wbase = wid * ROWS_PER_WORKER
    sems = [sem0, sem1, sem2, sem3]

    pltpu.sync_copy(ph_hbm.at[wid], ih_v.at[pl.ds(0, SET_CHUNKS)])
    pltpu.sync_copy(pr_hbm.at[wid], ir_v.at[pl.ds(0, SET_CHUNKS)])
    pltpu.sync_copy(pt_hbm.at[wid], it_v.at[pl.ds(0, SET_CHUNKS)])
    pltpu.sync_copy(nh_hbm.at[wid], ih_v.at[pl.ds(SET_CHUNKS, SET_CHUNKS)])
    pltpu.sync_copy(nr_hbm.at[wid], ir_v.at[pl.ds(SET_CHUNKS, SET_CHUNKS)])
    pltpu.sync_copy(nt_hbm.at[wid], it_v.at[pl.ds(SET_CHUNKS, SET_CHUNKS)])

    def fire(g, s):
        sem = sems[s]
        sl = pl.ds(s * CHUNK, CHUNK)
        return (
            pltpu.async_copy(ent_hbm.at[ih_v.at[g]], h_v.at[sl], sem),
            pltpu.async_copy(rel_hbm.at[ir_v.at[g]], r_v.at[sl], sem),
            pltpu.async_copy(ent_hbm.at[it_v.at[g]], t_v.at[sl], sem),
        )

    inflight = {}
    for g in range(NBUF):
        inflight[g] = fire(g, g % NBUF)

    for g in range(TOTAL_CHUNKS):
        s = g % NBUF
        for cp in inflight.pop(g):
            cp.wait()
        base = s * CHUNK

        def row_body(i, carry):
            for k in range(EMBED_DIM // 16):
                sl = pl.ds(k * 16, 16)
                h_v[base + i, sl] = jnp.abs(
                    h_v[base + i, sl] + r_v[base + i, sl] - t_v[base + i, sl])
            return carry

        lax.fori_loop(0, CHUNK, row_body, 0, unroll=4)

        out_hbm = pos_out if g < SET_CHUNKS else neg_out
        row0 = wbase + (g % SET_CHUNKS) * CHUNK
        pltpu.sync_copy(h_v.at[pl.ds(base, CHUNK)],
                        out_hbm.at[pl.ds(row0, CHUNK)])
        if g + NBUF < TOTAL_CHUNKS:
            inflight[g + NBUF] = fire(g + NBUF, s)


@jax.jit
def kernel(positive_samples, negative_samples, entity_embedding, relation_embedding):
    idx_shape = (NUM_WORKERS, SET_CHUNKS, CHUNK)
    ph = positive_samples[:, 0].reshape(idx_shape)
    pr = positive_samples[:, 1].reshape(idx_shape)
    pt = positive_samples[:, 2].reshape(idx_shape)
    nh = negative_samples[:, 0].reshape(idx_shape)
    nr = negative_samples[:, 1].reshape(idx_shape)
    nt = negative_samples[:, 2].reshape(idx_shape)

    mesh = plsc.VectorSubcoreMesh(core_axis_name="c", subcore_axis_name="s")
    out_t = jax.ShapeDtypeStruct((BATCH, EMBED_DIM), jnp.float32)
    run = pl.kernel(
        _transe_kernel,
        out_type=(out_t, out_t),
        mesh=mesh,
        compiler_params=pltpu.CompilerParams(use_tc_tiling_on_sc=False),
        scratch_types=[
            pltpu.VMEM((TOTAL_CHUNKS, CHUNK), jnp.int32),
            pltpu.VMEM((TOTAL_CHUNKS, CHUNK), jnp.int32),
            pltpu.VMEM((TOTAL_CHUNKS, CHUNK), jnp.int32),
            pltpu.VMEM((NBUF * CHUNK, EMBED_DIM), jnp.float32),
            pltpu.VMEM((NBUF * CHUNK, EMBED_DIM), jnp.float32),
            pltpu.VMEM((NBUF * CHUNK, EMBED_DIM), jnp.float32),
            pltpu.SemaphoreType.DMA,
            pltpu.SemaphoreType.DMA,
            pltpu.SemaphoreType.DMA,
            pltpu.SemaphoreType.DMA,
        ],
    )
    pos_out, neg_out = run(entity_embedding, relation_embedding,
                           ph, pr, pt, nh, nr, nt)
    return pos_out, neg_out
